# baseline (speedup 1.0000x reference)
import functools

import jax
import jax.numpy as jnp
from jax import lax
from jax.experimental import pallas as pl
from jax.experimental.pallas import tpu as pltpu, tpu_sc as plsc

S, C, L, T, B = 10, 100, 4, 6, 16384
TABLE_N = S * C * L * T

_info = plsc.get_sparse_core_info()
_NC, _NS, _LANES = _info.num_cores, _info.num_subcores, _info.num_lanes
_NW = _NC * _NS
_BPW = B // _NW
_STEPS = _BPW // _LANES

_mesh = plsc.VectorSubcoreMesh(core_axis_name="c", subcore_axis_name="s")


@functools.partial(
    pl.kernel,
    mesh=_mesh,
    out_type=jax.ShapeDtypeStruct((B,), jnp.int32),
    compiler_params=pltpu.CompilerParams(needs_layout_passes=False),
    scratch_types=[
        pltpu.VMEM((_BPW,), jnp.int32),
        pltpu.VMEM((_BPW,), jnp.int32),
        pltpu.VMEM((_BPW,), jnp.int32),
        pltpu.VMEM((_BPW,), jnp.int32),
        pltpu.VMEM((_BPW,), jnp.int32),
        pltpu.SemaphoreType.DMA,
    ],
)
def _linearize(src_hbm, cnt_hbm, lab_hbm, vt_hbm, lin_hbm,
               src_v, cnt_v, lab_v, vt_v, lin_v, sem):
    wid = lax.axis_index("s") * _NC + lax.axis_index("c")
    base = wid * _BPW

    sl_in = pl.ds(base, _BPW)
    copies = [
        pltpu.async_copy(src_hbm.at[sl_in], src_v, sem),
        pltpu.async_copy(cnt_hbm.at[sl_in], cnt_v, sem),
        pltpu.async_copy(lab_hbm.at[sl_in], lab_v, sem),
        pltpu.async_copy(vt_hbm.at[sl_in], vt_v, sem),
    ]
    for cp in copies:
        cp.wait()

    for i in range(_STEPS):
        sl = pl.ds(i * _LANES, _LANES)
        lin_v[sl] = (src_v[sl] * (C * L * T) + cnt_v[sl] * (L * T)
                     + lab_v[sl] * T + vt_v[sl])

    pltpu.sync_copy(lin_v, lin_hbm.at[sl_in])


@functools.partial(
    pl.kernel,
    mesh=_mesh,
    out_type=jax.ShapeDtypeStruct((B,), jnp.float32),
    compiler_params=pltpu.CompilerParams(needs_layout_passes=False),
    scratch_types=[
        pltpu.VMEM((_BPW,), jnp.int32),
        pltpu.VMEM((_BPW,), jnp.float32),
        pltpu.SemaphoreType.DMA,
    ],
)
def _gather(table_hbm, lin_hbm, out_hbm, lin_v, out_v, sem):
    wid = lax.axis_index("s") * _NC + lax.axis_index("c")
    base = wid * _BPW

    sl = pl.ds(base, _BPW)
    pltpu.sync_copy(lin_hbm.at[sl], lin_v)
    pltpu.async_copy(table_hbm.at[lin_v], out_v, sem).wait()
    pltpu.sync_copy(out_v, out_hbm.at[sl])


def kernel(label_balancing_weights_sclt, sources, alt_counts, labels, variant_types):
    lin = _linearize(sources, alt_counts, labels, variant_types)
    table = label_balancing_weights_sclt.reshape(-1)
    return _gather(table, lin)


# speedup vs baseline: 1.0908x; 1.0908x over previous
import functools

import jax
import jax.numpy as jnp
from jax import lax
from jax.experimental import pallas as pl
from jax.experimental.pallas import tpu as pltpu, tpu_sc as plsc

S, C, L, T, B = 10, 100, 4, 6, 16384
TABLE_N = S * C * L * T

_info = plsc.get_sparse_core_info()
_NC, _NS, _LANES = _info.num_cores, _info.num_subcores, _info.num_lanes
_NW = _NC * _NS
_BPW = B // _NW
_STEPS = _BPW // _LANES

_mesh = plsc.VectorSubcoreMesh(core_axis_name="c", subcore_axis_name="s")


@functools.partial(
    pl.kernel,
    mesh=_mesh,
    out_type=jax.ShapeDtypeStruct((B,), jnp.float32),
    compiler_params=pltpu.CompilerParams(
        needs_layout_passes=False, use_tc_tiling_on_sc=False),
    scratch_types=[
        pltpu.VMEM((_BPW,), jnp.int32),
        pltpu.VMEM((_BPW,), jnp.int32),
        pltpu.VMEM((_BPW,), jnp.int32),
        pltpu.VMEM((_BPW,), jnp.int32),
        pltpu.VMEM((_BPW,), jnp.int32),
        pltpu.VMEM((_BPW,), jnp.float32),
        pltpu.SemaphoreType.DMA,
        pltpu.SemaphoreType.DMA,
        pltpu.SemaphoreType.DMA,
    ],
)
def _balancer_gather(table_hbm, src_hbm, cnt_hbm, lab_hbm, vt_hbm, out_hbm,
                     src_v, cnt_v, lab_v, vt_v, lin_v, out_v, sem, gsem0,
                     gsem1):
    wid = lax.axis_index("s") * _NC + lax.axis_index("c")
    base = wid * _BPW

    sl_in = pl.ds(base, _BPW)
    copies = [
        pltpu.async_copy(src_hbm.at[sl_in], src_v, sem),
        pltpu.async_copy(cnt_hbm.at[sl_in], cnt_v, sem),
        pltpu.async_copy(lab_hbm.at[sl_in], lab_v, sem),
        pltpu.async_copy(vt_hbm.at[sl_in], vt_v, sem),
    ]
    for cp in copies:
        cp.wait()

    half = _BPW // 2
    gsems = (gsem0, gsem1)
    gathers = []
    for h in range(2):
        for i in range(_STEPS // 2):
            sl = pl.ds(h * half + i * _LANES, _LANES)
            lin_v[sl] = (src_v[sl] * (C * L * T) + cnt_v[sl] * (L * T)
                         + lab_v[sl] * T + vt_v[sl])
        gathers.append(pltpu.async_copy(
            table_hbm.at[lin_v.at[pl.ds(h * half, half)]],
            out_v.at[pl.ds(h * half, half)], gsems[h]))

    outs = []
    for h in range(2):
        gathers[h].wait()
        outs.append(pltpu.async_copy(
            out_v.at[pl.ds(h * half, half)],
            out_hbm.at[pl.ds(base + h * half, half)], sem))
    for cp in outs:
        cp.wait()


def kernel(label_balancing_weights_sclt, sources, alt_counts, labels, variant_types):
    table = label_balancing_weights_sclt.reshape(-1)
    return _balancer_gather(table, sources, alt_counts, labels, variant_types)
